# Initial kernel scaffold; baseline (speedup 1.0000x reference)
import jax
import jax.numpy as jnp
from jax.experimental import pallas as pl


def kernel(x, edge_index, batch, W1l, b1l, W1r, g1, be1, W2l, b2l, W2r, g2, be2, Wf, bf, Wf1, bf1):
    raise NotImplementedError("write your pallas kernel here")


# speedup vs baseline: 2.4366x; 2.4366x over previous
import jax
import jax.numpy as jnp
from jax import lax
from jax.experimental import pallas as pl
from jax.experimental.pallas import tpu as pltpu
from jax.experimental.pallas import tpu_sc as plsc

N = 10000
E = 320000
D_IN = 128
D_H = 1024
D_FC = 512
D_OUT = 128
EPS_BN = 1e-5

NP = 10112
RPT = NP // 16
K = 128
SUP = 8
NB1 = 80
NB2 = 160
RB = 512
GRID = (N + RB - 1) // RB

_f32 = jnp.float32
_mesh = plsc.VectorSubcoreMesh(core_axis_name="c", subcore_axis_name="s")


def _sc_l1_body(x_hbm, srcI, dstI, ones_hbm, zrow_hbm,
                aggp_hbm, cntp_hbm,
                srcb, dstb, rowbuf, onesv, aggS, sem):
    c = lax.axis_index("c")
    s = lax.axis_index("s")
    base = s * RPT
    irow0 = (c * 16 + s) * NB1
    pltpu.sync_copy(ones_hbm, onesv)
    pltpu.sync_copy(zrow_hbm, aggS.at[pl.ds(base, RPT)])
    plsc.subcore_barrier()

    def sup_body(g, carry):
        pltpu.sync_copy(srcI.at[pl.ds(irow0 + g * SUP, SUP)], srcb)
        pltpu.sync_copy(dstI.at[pl.ds(irow0 + g * SUP, SUP)], dstb)

        def blk(bi, carry2):
            pltpu.async_copy(x_hbm.at[srcb.at[bi]], rowbuf, sem).wait()
            pltpu.sync_copy(rowbuf, aggS.at[dstb.at[bi]], add=True)
            return carry2

        lax.fori_loop(0, SUP, blk, 0)
        return carry

    lax.fori_loop(0, NB1 // SUP, sup_body, 0)
    plsc.subcore_barrier()
    pltpu.sync_copy(aggS.at[pl.ds(base, RPT)],
                    aggp_hbm.at[pl.ds(c * NP + base, RPT)])
    pltpu.sync_copy(zrow_hbm, aggS.at[pl.ds(base, RPT)])
    plsc.subcore_barrier()

    def cnt_body(g, carry):
        pltpu.sync_copy(dstI.at[pl.ds(irow0 + g * SUP, SUP)], dstb)

        def blk(bi, carry2):
            pltpu.sync_copy(onesv, aggS.at[dstb.at[bi]], add=True)
            return carry2

        lax.fori_loop(0, SUP, blk, 0)
        return carry

    lax.fori_loop(0, NB1 // SUP, cnt_body, 0)
    plsc.subcore_barrier()
    pltpu.sync_copy(aggS.at[pl.ds(base, RPT)],
                    cntp_hbm.at[pl.ds(c * NP + base, RPT)])


_sc_l1 = pl.kernel(
    _sc_l1_body,
    out_type=(jax.ShapeDtypeStruct((2 * NP, D_IN), _f32),
              jax.ShapeDtypeStruct((2 * NP, 128), _f32)),
    mesh=_mesh,
    scratch_types=[
        pltpu.VMEM((SUP, K), jnp.int32),
        pltpu.VMEM((SUP, K), jnp.int32),
        pltpu.VMEM((K, D_IN), _f32),
        pltpu.VMEM((K, 128), _f32),
        pltpu.VMEM_SHARED((NP, D_IN), _f32),
        pltpu.SemaphoreType.DMA,
    ],
)


def _sc_l2_body(h1f_hbm, srcI2, dstI2, zrow_hbm,
                agg2_hbm,
                srcb, dstb, rowbuf, aggS, sem):
    c = lax.axis_index("c")
    s = lax.axis_index("s")
    base = s * RPT
    for j in range(4):
        ci = 2 * j + c
        irow0 = (ci * 16 + s) * NB2
        drow0 = s * NB2
        pltpu.sync_copy(zrow_hbm, aggS.at[pl.ds(base, RPT)])
        plsc.subcore_barrier()

        def sup_body(g, carry):
            pltpu.sync_copy(srcI2.at[pl.ds(irow0 + g * SUP, SUP)], srcb)
            pltpu.sync_copy(dstI2.at[pl.ds(drow0 + g * SUP, SUP)], dstb)

            def blk(bi, carry2):
                pltpu.async_copy(h1f_hbm.at[srcb.at[bi]], rowbuf, sem).wait()
                pltpu.sync_copy(rowbuf, aggS.at[dstb.at[bi]], add=True)
                return carry2

            lax.fori_loop(0, SUP, blk, 0)
            return carry

        lax.fori_loop(0, NB2 // SUP, sup_body, 0)
        plsc.subcore_barrier()
        pltpu.sync_copy(aggS.at[pl.ds(base, RPT)],
                        agg2_hbm.at[pl.ds(ci * NP + base, RPT)])


_sc_l2 = pl.kernel(
    _sc_l2_body,
    out_type=jax.ShapeDtypeStruct((8 * NP, 128), _f32),
    mesh=_mesh,
    scratch_types=[
        pltpu.VMEM((SUP, K), jnp.int32),
        pltpu.VMEM((SUP, K), jnp.int32),
        pltpu.VMEM((K, 128), _f32),
        pltpu.VMEM_SHARED((NP, 128), _f32),
        pltpu.SemaphoreType.DMA,
    ],
)


def _a1_body(aggp, cntp, x, wl, wr, b, z_ref, st_ref):
    i = pl.program_id(0)
    agg = aggp[0] + aggp[1]
    cnt = cntp[0, :, 0:1] + cntp[1, :, 0:1]
    inv = 1.0 / jnp.maximum(cnt, 1.0)
    z = (jnp.dot(agg * inv, wl[...], preferred_element_type=_f32)
         + jnp.dot(x[...], wr[...], preferred_element_type=_f32)
         + b[...])
    z_ref[...] = z

    @pl.when(i == 0)
    def _():
        st_ref[...] = jnp.zeros_like(st_ref)

    rid = lax.broadcasted_iota(jnp.int32, (RB, 1), 0) + i * RB
    zm = jnp.where(rid < N, z, 0.0)
    st_ref[0:1, :] += jnp.sum(zm, axis=0, keepdims=True)
    st_ref[1:2, :] += jnp.sum(zm * zm, axis=0, keepdims=True)


def _bn_coefs(st, g, be):
    mu = st[0:1, :] * (1.0 / N)
    var = st[1:2, :] * (1.0 / N) - mu * mu
    scale = g[...] * lax.rsqrt(var + EPS_BN)
    shift = be[...] - mu * scale
    return scale, shift


def _b1_body(z, st, g, be, out):
    scale, shift = _bn_coefs(st, g, be)
    h = jnp.maximum(z[...] * scale + shift, 0.0)
    for ci in range(8):
        out[ci] = h[:, ci * 128:(ci + 1) * 128]


def _a2_body(agg2, cntp, h1cm, wl, wr, b, z_ref, st_ref):
    i = pl.program_id(0)
    cnt = cntp[0, :, 0:1] + cntp[1, :, 0:1]
    inv = 1.0 / jnp.maximum(cnt, 1.0)
    sagg = jnp.concatenate([agg2[ci] for ci in range(8)], axis=1)
    hc = jnp.concatenate([h1cm[ci] for ci in range(8)], axis=1)
    z = (jnp.dot(sagg * inv, wl[...], preferred_element_type=_f32)
         + jnp.dot(hc, wr[...], preferred_element_type=_f32)
         + b[...])
    z_ref[...] = z

    @pl.when(i == 0)
    def _():
        st_ref[...] = jnp.zeros_like(st_ref)

    rid = lax.broadcasted_iota(jnp.int32, (RB, 1), 0) + i * RB
    zm = jnp.where(rid < N, z, 0.0)
    st_ref[0:1, :] += jnp.sum(zm, axis=0, keepdims=True)
    st_ref[1:2, :] += jnp.sum(zm * zm, axis=0, keepdims=True)


def _bc_body(z, st, g, be, wf, bf, wf1, bf1, out):
    scale, shift = _bn_coefs(st, g, be)
    h2 = jnp.maximum(z[...] * scale + shift, 0.0)
    h3 = jnp.maximum(
        jnp.dot(h2, wf[...], preferred_element_type=_f32) + bf[...], 0.0)
    out[...] = jnp.dot(h3, wf1[...], preferred_element_type=_f32) + bf1[...]


def _full(shape):
    return pl.BlockSpec(shape, lambda i: tuple(0 for _ in shape))


_a1 = pl.pallas_call(
    _a1_body,
    grid=(GRID,),
    in_specs=[
        pl.BlockSpec((2, RB, D_IN), lambda i: (0, i, 0)),
        pl.BlockSpec((2, RB, 128), lambda i: (0, i, 0)),
        pl.BlockSpec((RB, D_IN), lambda i: (i, 0)),
        _full((D_IN, D_H)),
        _full((D_IN, D_H)),
        _full((1, D_H)),
    ],
    out_specs=[
        pl.BlockSpec((RB, D_H), lambda i: (i, 0)),
        pl.BlockSpec((8, D_H), lambda i: (0, 0)),
    ],
    out_shape=[
        jax.ShapeDtypeStruct((N, D_H), _f32),
        jax.ShapeDtypeStruct((8, D_H), _f32),
    ],
)

_b1 = pl.pallas_call(
    _b1_body,
    grid=(GRID,),
    in_specs=[
        pl.BlockSpec((RB, D_H), lambda i: (i, 0)),
        _full((8, D_H)),
        _full((1, D_H)),
        _full((1, D_H)),
    ],
    out_specs=pl.BlockSpec((8, RB, 128), lambda i: (0, i, 0)),
    out_shape=jax.ShapeDtypeStruct((8, N, 128), _f32),
)

_a2 = pl.pallas_call(
    _a2_body,
    grid=(GRID,),
    in_specs=[
        pl.BlockSpec((8, RB, 128), lambda i: (0, i, 0)),
        pl.BlockSpec((2, RB, 128), lambda i: (0, i, 0)),
        pl.BlockSpec((8, RB, 128), lambda i: (0, i, 0)),
        _full((D_H, D_H)),
        _full((D_H, D_H)),
        _full((1, D_H)),
    ],
    out_specs=[
        pl.BlockSpec((RB, D_H), lambda i: (i, 0)),
        pl.BlockSpec((8, D_H), lambda i: (0, 0)),
    ],
    out_shape=[
        jax.ShapeDtypeStruct((N, D_H), _f32),
        jax.ShapeDtypeStruct((8, D_H), _f32),
    ],
)

_bc = pl.pallas_call(
    _bc_body,
    grid=(GRID,),
    in_specs=[
        pl.BlockSpec((RB, D_H), lambda i: (i, 0)),
        _full((8, D_H)),
        _full((1, D_H)),
        _full((1, D_H)),
        _full((D_H, D_FC)),
        _full((1, D_FC)),
        _full((D_FC, D_OUT)),
        _full((1, D_OUT)),
    ],
    out_specs=pl.BlockSpec((RB, D_OUT), lambda i: (i, 0)),
    out_shape=jax.ShapeDtypeStruct((N, D_OUT), _f32),
)


def kernel(x, edge_index, batch, W1l, b1l, W1r, g1, be1,
           W2l, b2l, W2r, g2, be2, Wf, bf, Wf1, bf1):
    src = edge_index[0].astype(jnp.int32)
    dst = edge_index[1].astype(jnp.int32)

    pad1 = NB1 * K - E // 32
    srcI1 = jnp.concatenate(
        [src.reshape(32, -1), jnp.zeros((32, pad1), jnp.int32)],
        axis=1).reshape(32 * NB1, K)
    dstI1 = jnp.concatenate(
        [dst.reshape(32, -1), jnp.full((32, pad1), N, jnp.int32)],
        axis=1).reshape(32 * NB1, K)

    pad2 = NB2 * K - E // 16
    src16 = jnp.concatenate(
        [src.reshape(16, -1), jnp.zeros((16, pad2), jnp.int32)], axis=1)
    ci_off = (jnp.arange(8) * N).astype(jnp.int32)
    srcI2 = (src16[None] + ci_off[:, None, None]).reshape(8 * 16 * NB2, K)
    dstI2 = jnp.concatenate(
        [dst.reshape(16, -1), jnp.full((16, pad2), N, jnp.int32)],
        axis=1).reshape(16 * NB2, K)

    ones = jnp.ones((K, 128), _f32)
    zrow = jnp.zeros((RPT, 128), _f32)

    aggpf, cntpf = _sc_l1(x, srcI1, dstI1, ones, zrow)
    aggp = aggpf.reshape(2, NP, D_IN)
    cntp = cntpf.reshape(2, NP, 128)
    z1, st1 = _a1(aggp, cntp, x, W1l, W1r, b1l.reshape(1, -1))
    h1cm = _b1(z1, st1, g1.reshape(1, -1), be1.reshape(1, -1))
    h1f = h1cm.reshape(8 * N, 128)
    agg2 = _sc_l2(h1f, srcI2, dstI2, zrow).reshape(8, NP, 128)
    z2, st2 = _a2(agg2, cntp, h1cm, W2l, W2r, b2l.reshape(1, -1))
    out = _bc(z2, st2, g2.reshape(1, -1), be2.reshape(1, -1),
              Wf, bf.reshape(1, -1), Wf1, bf1.reshape(1, -1))
    return out


# speedup vs baseline: 2.8016x; 1.1498x over previous
import jax
import jax.numpy as jnp
from jax import lax
from jax.experimental import pallas as pl
from jax.experimental.pallas import tpu as pltpu
from jax.experimental.pallas import tpu_sc as plsc

N = 10000
E = 320000
D_IN = 128
D_H = 1024
D_FC = 512
D_OUT = 128
EPS_BN = 1e-5

NP = 10112
RPT = NP // 16
K = 128
SUP = 8
NB1 = 80
NB2 = 160
RB = 512
GRID = (N + RB - 1) // RB

_f32 = jnp.float32
_mesh = plsc.VectorSubcoreMesh(core_axis_name="c", subcore_axis_name="s")


def _sc_l1_body(x_hbm, srcI, dstI, ones_hbm, zrow_hbm,
                aggp_hbm, cntp_hbm,
                srcb, dstb, rowbuf, onesv, aggS, sem):
    c = lax.axis_index("c")
    s = lax.axis_index("s")
    base = s * RPT
    irow0 = (c * 16 + s) * NB1
    pltpu.sync_copy(ones_hbm, onesv)
    pltpu.sync_copy(zrow_hbm, aggS.at[pl.ds(base, RPT)])
    plsc.subcore_barrier()

    def sup_body(g, carry):
        pltpu.sync_copy(srcI.at[pl.ds(irow0 + g * SUP, SUP)], srcb)
        pltpu.sync_copy(dstI.at[pl.ds(irow0 + g * SUP, SUP)], dstb)

        def blk(bi, carry2):
            pltpu.async_copy(x_hbm.at[srcb.at[bi]], rowbuf, sem).wait()
            pltpu.sync_copy(rowbuf, aggS.at[dstb.at[bi]], add=True)
            return carry2

        lax.fori_loop(0, SUP, blk, 0)
        return carry

    lax.fori_loop(0, NB1 // SUP, sup_body, 0)
    plsc.subcore_barrier()
    pltpu.sync_copy(aggS.at[pl.ds(base, RPT)],
                    aggp_hbm.at[pl.ds(c * NP + base, RPT)])
    pltpu.sync_copy(zrow_hbm, aggS.at[pl.ds(base, RPT)])
    plsc.subcore_barrier()

    def cnt_body(g, carry):
        pltpu.sync_copy(dstI.at[pl.ds(irow0 + g * SUP, SUP)], dstb)
        ds_ = [pltpu.async_copy(onesv, aggS.at[dstb.at[bi]], sem, add=True)
               for bi in range(SUP)]
        for d in ds_:
            d.wait()
        return carry

    lax.fori_loop(0, NB1 // SUP, cnt_body, 0)
    plsc.subcore_barrier()
    pltpu.sync_copy(aggS.at[pl.ds(base, RPT)],
                    cntp_hbm.at[pl.ds(c * NP + base, RPT)])


_sc_l1 = pl.kernel(
    _sc_l1_body,
    out_type=(jax.ShapeDtypeStruct((2 * NP, D_IN), _f32),
              jax.ShapeDtypeStruct((2 * NP, 128), _f32)),
    mesh=_mesh,
    scratch_types=[
        pltpu.VMEM((SUP, K), jnp.int32),
        pltpu.VMEM((SUP, K), jnp.int32),
        pltpu.VMEM((K, D_IN), _f32),
        pltpu.VMEM((K, 128), _f32),
        pltpu.VMEM_SHARED((NP, D_IN), _f32),
        pltpu.SemaphoreType.DMA,
    ],
)


def _sc_l2_body(h1f_hbm, srcI2, dstI2, zrow_hbm,
                agg2_hbm,
                srcb, dstb, rowbuf0, rowbuf1, aggS,
                semg0, semg1, sems0, sems1):
    c = lax.axis_index("c")
    s = lax.axis_index("s")
    base = s * RPT
    bufs = (rowbuf0, rowbuf1)
    semg = (semg0, semg1)
    sems = (sems0, sems1)
    for j in range(4):
        ci = 2 * j + c
        irow0 = (ci * 16 + s) * NB2
        drow0 = s * NB2
        pltpu.sync_copy(zrow_hbm, aggS.at[pl.ds(base, RPT)])
        plsc.subcore_barrier()

        def sup_body(g, carry):
            pltpu.sync_copy(srcI2.at[pl.ds(irow0 + g * SUP, SUP)], srcb)
            pltpu.sync_copy(dstI2.at[pl.ds(drow0 + g * SUP, SUP)], dstb)
            gd = [None, None]
            sd = [None, None]
            gd[0] = pltpu.async_copy(h1f_hbm.at[srcb.at[0]], bufs[0], semg[0])
            for bi in range(SUP):
                cur = bi % 2
                nxt = (bi + 1) % 2
                if bi + 1 < SUP:
                    if sd[nxt] is not None:
                        sd[nxt].wait()
                    gd[nxt] = pltpu.async_copy(
                        h1f_hbm.at[srcb.at[bi + 1]], bufs[nxt], semg[nxt])
                gd[cur].wait()
                sd[cur] = pltpu.async_copy(
                    bufs[cur], aggS.at[dstb.at[bi]], sems[cur], add=True)
            sd[0].wait()
            sd[1].wait()
            return carry

        lax.fori_loop(0, NB2 // SUP, sup_body, 0)
        plsc.subcore_barrier()
        pltpu.sync_copy(aggS.at[pl.ds(base, RPT)],
                        agg2_hbm.at[pl.ds(ci * NP + base, RPT)])


_sc_l2 = pl.kernel(
    _sc_l2_body,
    out_type=jax.ShapeDtypeStruct((8 * NP, 128), _f32),
    mesh=_mesh,
    scratch_types=[
        pltpu.VMEM((SUP, K), jnp.int32),
        pltpu.VMEM((SUP, K), jnp.int32),
        pltpu.VMEM((K, 128), _f32),
        pltpu.VMEM((K, 128), _f32),
        pltpu.VMEM_SHARED((NP, 128), _f32),
        pltpu.SemaphoreType.DMA,
        pltpu.SemaphoreType.DMA,
        pltpu.SemaphoreType.DMA,
        pltpu.SemaphoreType.DMA,
    ],
)


def _a1_body(aggp, cntp, x, wl, wr, b, z_ref, st_ref):
    i = pl.program_id(0)
    agg = aggp[0] + aggp[1]
    cnt = cntp[0, :, 0:1] + cntp[1, :, 0:1]
    inv = 1.0 / jnp.maximum(cnt, 1.0)
    z = (jnp.dot(agg * inv, wl[...], preferred_element_type=_f32)
         + jnp.dot(x[...], wr[...], preferred_element_type=_f32)
         + b[...])
    z_ref[...] = z

    @pl.when(i == 0)
    def _():
        st_ref[...] = jnp.zeros_like(st_ref)

    rid = lax.broadcasted_iota(jnp.int32, (RB, 1), 0) + i * RB
    zm = jnp.where(rid < N, z, 0.0)
    st_ref[0:1, :] += jnp.sum(zm, axis=0, keepdims=True)
    st_ref[1:2, :] += jnp.sum(zm * zm, axis=0, keepdims=True)


def _bn_coefs(st, g, be):
    mu = st[0:1, :] * (1.0 / N)
    var = st[1:2, :] * (1.0 / N) - mu * mu
    scale = g[...] * lax.rsqrt(var + EPS_BN)
    shift = be[...] - mu * scale
    return scale, shift


def _b1_body(z, st, g, be, out):
    scale, shift = _bn_coefs(st, g, be)
    h = jnp.maximum(z[...] * scale + shift, 0.0)
    for ci in range(8):
        out[ci] = h[:, ci * 128:(ci + 1) * 128]


def _a2_body(agg2, cntp, h1cm, wl, wr, b, z_ref, st_ref):
    i = pl.program_id(0)
    cnt = cntp[0, :, 0:1] + cntp[1, :, 0:1]
    inv = 1.0 / jnp.maximum(cnt, 1.0)
    sagg = jnp.concatenate([agg2[ci] for ci in range(8)], axis=1)
    hc = jnp.concatenate([h1cm[ci] for ci in range(8)], axis=1)
    z = (jnp.dot(sagg * inv, wl[...], preferred_element_type=_f32)
         + jnp.dot(hc, wr[...], preferred_element_type=_f32)
         + b[...])
    z_ref[...] = z

    @pl.when(i == 0)
    def _():
        st_ref[...] = jnp.zeros_like(st_ref)

    rid = lax.broadcasted_iota(jnp.int32, (RB, 1), 0) + i * RB
    zm = jnp.where(rid < N, z, 0.0)
    st_ref[0:1, :] += jnp.sum(zm, axis=0, keepdims=True)
    st_ref[1:2, :] += jnp.sum(zm * zm, axis=0, keepdims=True)


def _bc_body(z, st, g, be, wf, bf, wf1, bf1, out):
    scale, shift = _bn_coefs(st, g, be)
    h2 = jnp.maximum(z[...] * scale + shift, 0.0)
    h3 = jnp.maximum(
        jnp.dot(h2, wf[...], preferred_element_type=_f32) + bf[...], 0.0)
    out[...] = jnp.dot(h3, wf1[...], preferred_element_type=_f32) + bf1[...]


def _full(shape):
    return pl.BlockSpec(shape, lambda i: tuple(0 for _ in shape))


_a1 = pl.pallas_call(
    _a1_body,
    grid=(GRID,),
    in_specs=[
        pl.BlockSpec((2, RB, D_IN), lambda i: (0, i, 0)),
        pl.BlockSpec((2, RB, 128), lambda i: (0, i, 0)),
        pl.BlockSpec((RB, D_IN), lambda i: (i, 0)),
        _full((D_IN, D_H)),
        _full((D_IN, D_H)),
        _full((1, D_H)),
    ],
    out_specs=[
        pl.BlockSpec((RB, D_H), lambda i: (i, 0)),
        pl.BlockSpec((8, D_H), lambda i: (0, 0)),
    ],
    out_shape=[
        jax.ShapeDtypeStruct((N, D_H), _f32),
        jax.ShapeDtypeStruct((8, D_H), _f32),
    ],
)

_b1 = pl.pallas_call(
    _b1_body,
    grid=(GRID,),
    in_specs=[
        pl.BlockSpec((RB, D_H), lambda i: (i, 0)),
        _full((8, D_H)),
        _full((1, D_H)),
        _full((1, D_H)),
    ],
    out_specs=pl.BlockSpec((8, RB, 128), lambda i: (0, i, 0)),
    out_shape=jax.ShapeDtypeStruct((8, N, 128), _f32),
)

_a2 = pl.pallas_call(
    _a2_body,
    grid=(GRID,),
    in_specs=[
        pl.BlockSpec((8, RB, 128), lambda i: (0, i, 0)),
        pl.BlockSpec((2, RB, 128), lambda i: (0, i, 0)),
        pl.BlockSpec((8, RB, 128), lambda i: (0, i, 0)),
        _full((D_H, D_H)),
        _full((D_H, D_H)),
        _full((1, D_H)),
    ],
    out_specs=[
        pl.BlockSpec((RB, D_H), lambda i: (i, 0)),
        pl.BlockSpec((8, D_H), lambda i: (0, 0)),
    ],
    out_shape=[
        jax.ShapeDtypeStruct((N, D_H), _f32),
        jax.ShapeDtypeStruct((8, D_H), _f32),
    ],
)

_bc = pl.pallas_call(
    _bc_body,
    grid=(GRID,),
    in_specs=[
        pl.BlockSpec((RB, D_H), lambda i: (i, 0)),
        _full((8, D_H)),
        _full((1, D_H)),
        _full((1, D_H)),
        _full((D_H, D_FC)),
        _full((1, D_FC)),
        _full((D_FC, D_OUT)),
        _full((1, D_OUT)),
    ],
    out_specs=pl.BlockSpec((RB, D_OUT), lambda i: (i, 0)),
    out_shape=jax.ShapeDtypeStruct((N, D_OUT), _f32),
)


def kernel(x, edge_index, batch, W1l, b1l, W1r, g1, be1,
           W2l, b2l, W2r, g2, be2, Wf, bf, Wf1, bf1):
    src = edge_index[0].astype(jnp.int32)
    dst = edge_index[1].astype(jnp.int32)

    pad1 = NB1 * K - E // 32
    srcI1 = jnp.concatenate(
        [src.reshape(32, -1), jnp.zeros((32, pad1), jnp.int32)],
        axis=1).reshape(32 * NB1, K)
    dstI1 = jnp.concatenate(
        [dst.reshape(32, -1), jnp.full((32, pad1), N, jnp.int32)],
        axis=1).reshape(32 * NB1, K)

    pad2 = NB2 * K - E // 16
    src16 = jnp.concatenate(
        [src.reshape(16, -1), jnp.zeros((16, pad2), jnp.int32)], axis=1)
    ci_off = (jnp.arange(8) * N).astype(jnp.int32)
    srcI2 = (src16[None] + ci_off[:, None, None]).reshape(8 * 16 * NB2, K)
    dstI2 = jnp.concatenate(
        [dst.reshape(16, -1), jnp.full((16, pad2), N, jnp.int32)],
        axis=1).reshape(16 * NB2, K)

    ones = jnp.ones((K, 128), _f32)
    zrow = jnp.zeros((RPT, 128), _f32)

    aggpf, cntpf = _sc_l1(x, srcI1, dstI1, ones, zrow)
    aggp = aggpf.reshape(2, NP, D_IN)
    cntp = cntpf.reshape(2, NP, 128)
    z1, st1 = _a1(aggp, cntp, x, W1l, W1r, b1l.reshape(1, -1))
    h1cm = _b1(z1, st1, g1.reshape(1, -1), be1.reshape(1, -1))
    h1f = h1cm.reshape(8 * N, 128)
    agg2 = _sc_l2(h1f, srcI2, dstI2, zrow).reshape(8, NP, 128)
    z2, st2 = _a2(agg2, cntp, h1cm, W2l, W2r, b2l.reshape(1, -1))
    out = _bc(z2, st2, g2.reshape(1, -1), be2.reshape(1, -1),
              Wf, bf.reshape(1, -1), Wf1, bf1.reshape(1, -1))
    return out


# speedup vs baseline: 2.8522x; 1.0181x over previous
import jax
import jax.numpy as jnp
from jax import lax
from jax.experimental import pallas as pl
from jax.experimental.pallas import tpu as pltpu
from jax.experimental.pallas import tpu_sc as plsc

N = 10000
E = 320000
D_IN = 128
D_H = 1024
D_FC = 512
D_OUT = 128
EPS_BN = 1e-5

NP = 10112
RPT = NP // 16
K = 128
SUP = 8
NB1 = 80
NB2 = 160
RB = 512
GRID = (N + RB - 1) // RB

_f32 = jnp.float32
_mesh = plsc.VectorSubcoreMesh(core_axis_name="c", subcore_axis_name="s")


def _sc_l1_body(x_hbm, srcI, dstI, ones_hbm, zrow_hbm,
                aggp_hbm, cntp_hbm,
                srcb, dstb, rowbuf0, rowbuf1, aggS,
                semg0, semg1, sems0, sems1):
    c = lax.axis_index("c")
    s = lax.axis_index("s")
    base = s * RPT
    irow0 = (c * 16 + s) * NB1
    bufs = (rowbuf0, rowbuf1)
    semg = (semg0, semg1)
    sems = (sems0, sems1)
    pltpu.sync_copy(zrow_hbm, aggS.at[pl.ds(base, RPT)])
    plsc.subcore_barrier()

    def sup_body(g, carry):
        pltpu.sync_copy(srcI.at[pl.ds(irow0 + g * SUP, SUP)], srcb)
        pltpu.sync_copy(dstI.at[pl.ds(irow0 + g * SUP, SUP)], dstb)
        gd = [None, None]
        sd = [None, None]
        gd[0] = pltpu.async_copy(x_hbm.at[srcb.at[0]], bufs[0], semg[0])
        for bi in range(SUP):
            cur = bi % 2
            nxt = (bi + 1) % 2
            if bi + 1 < SUP:
                if sd[nxt] is not None:
                    sd[nxt].wait()
                gd[nxt] = pltpu.async_copy(
                    x_hbm.at[srcb.at[bi + 1]], bufs[nxt], semg[nxt])
            gd[cur].wait()
            sd[cur] = pltpu.async_copy(
                bufs[cur], aggS.at[dstb.at[bi]], sems[cur], add=True)
        sd[0].wait()
        sd[1].wait()
        return carry

    lax.fori_loop(0, NB1 // SUP, sup_body, 0)
    plsc.subcore_barrier()
    pltpu.sync_copy(aggS.at[pl.ds(base, RPT)],
                    aggp_hbm.at[pl.ds(c * NP + base, RPT)])
    pltpu.sync_copy(zrow_hbm, aggS.at[pl.ds(base, RPT)])
    pltpu.sync_copy(ones_hbm, rowbuf1)
    plsc.subcore_barrier()

    def cnt_body(g, carry):
        pltpu.sync_copy(dstI.at[pl.ds(irow0 + g * SUP, SUP)], dstb)
        ds_ = [pltpu.async_copy(rowbuf1, aggS.at[dstb.at[bi]],
                                sems[bi % 2], add=True)
               for bi in range(SUP)]
        for d in ds_:
            d.wait()
        return carry

    lax.fori_loop(0, NB1 // SUP, cnt_body, 0)
    plsc.subcore_barrier()
    pltpu.sync_copy(aggS.at[pl.ds(base, RPT)],
                    cntp_hbm.at[pl.ds(c * NP + base, RPT)])


_sc_l1 = pl.kernel(
    _sc_l1_body,
    out_type=(jax.ShapeDtypeStruct((2 * NP, D_IN), _f32),
              jax.ShapeDtypeStruct((2 * NP, 128), _f32)),
    mesh=_mesh,
    scratch_types=[
        pltpu.VMEM((SUP, K), jnp.int32),
        pltpu.VMEM((SUP, K), jnp.int32),
        pltpu.VMEM((K, D_IN), _f32),
        pltpu.VMEM((K, 128), _f32),
        pltpu.VMEM_SHARED((NP, D_IN), _f32),
        pltpu.SemaphoreType.DMA,
        pltpu.SemaphoreType.DMA,
        pltpu.SemaphoreType.DMA,
        pltpu.SemaphoreType.DMA,
    ],
)


def _sc_l2_body(h1f_hbm, srcI2, dstI2, zrow_hbm,
                agg2_hbm,
                srcb, dstb, rowbuf0, rowbuf1, aggS,
                semg0, semg1, sems0, sems1):
    c = lax.axis_index("c")
    s = lax.axis_index("s")
    base = s * RPT
    bufs = (rowbuf0, rowbuf1)
    semg = (semg0, semg1)
    sems = (sems0, sems1)
    for j in range(4):
        ci = 2 * j + c
        irow0 = (ci * 16 + s) * NB2
        drow0 = s * NB2
        pltpu.sync_copy(zrow_hbm, aggS.at[pl.ds(base, RPT)])
        plsc.subcore_barrier()

        def sup_body(g, carry):
            pltpu.sync_copy(srcI2.at[pl.ds(irow0 + g * SUP, SUP)], srcb)
            pltpu.sync_copy(dstI2.at[pl.ds(drow0 + g * SUP, SUP)], dstb)
            gd = [None, None]
            sd = [None, None]
            gd[0] = pltpu.async_copy(h1f_hbm.at[srcb.at[0]], bufs[0], semg[0])
            for bi in range(SUP):
                cur = bi % 2
                nxt = (bi + 1) % 2
                if bi + 1 < SUP:
                    if sd[nxt] is not None:
                        sd[nxt].wait()
                    gd[nxt] = pltpu.async_copy(
                        h1f_hbm.at[srcb.at[bi + 1]], bufs[nxt], semg[nxt])
                gd[cur].wait()
                sd[cur] = pltpu.async_copy(
                    bufs[cur], aggS.at[dstb.at[bi]], sems[cur], add=True)
            sd[0].wait()
            sd[1].wait()
            return carry

        lax.fori_loop(0, NB2 // SUP, sup_body, 0)
        plsc.subcore_barrier()
        pltpu.sync_copy(aggS.at[pl.ds(base, RPT)],
                        agg2_hbm.at[pl.ds(ci * NP + base, RPT)])


_sc_l2 = pl.kernel(
    _sc_l2_body,
    out_type=jax.ShapeDtypeStruct((8 * NP, 128), _f32),
    mesh=_mesh,
    scratch_types=[
        pltpu.VMEM((SUP, K), jnp.int32),
        pltpu.VMEM((SUP, K), jnp.int32),
        pltpu.VMEM((K, 128), _f32),
        pltpu.VMEM((K, 128), _f32),
        pltpu.VMEM_SHARED((NP, 128), _f32),
        pltpu.SemaphoreType.DMA,
        pltpu.SemaphoreType.DMA,
        pltpu.SemaphoreType.DMA,
        pltpu.SemaphoreType.DMA,
    ],
)


def _a1_body(aggp, cntp, x, wl, wr, b, z_ref, st_ref):
    i = pl.program_id(0)
    agg = aggp[0] + aggp[1]
    cnt = cntp[0, :, 0:1] + cntp[1, :, 0:1]
    inv = 1.0 / jnp.maximum(cnt, 1.0)
    z = (jnp.dot(agg * inv, wl[...], preferred_element_type=_f32)
         + jnp.dot(x[...], wr[...], preferred_element_type=_f32)
         + b[...])
    z_ref[...] = z

    @pl.when(i == 0)
    def _():
        st_ref[...] = jnp.zeros_like(st_ref)

    rid = lax.broadcasted_iota(jnp.int32, (RB, 1), 0) + i * RB
    zm = jnp.where(rid < N, z, 0.0)
    st_ref[0:1, :] += jnp.sum(zm, axis=0, keepdims=True)
    st_ref[1:2, :] += jnp.sum(zm * zm, axis=0, keepdims=True)


def _bn_coefs(st, g, be):
    mu = st[0:1, :] * (1.0 / N)
    var = st[1:2, :] * (1.0 / N) - mu * mu
    scale = g[...] * lax.rsqrt(var + EPS_BN)
    shift = be[...] - mu * scale
    return scale, shift


def _b1_body(z, st, g, be, out):
    scale, shift = _bn_coefs(st, g, be)
    h = jnp.maximum(z[...] * scale + shift, 0.0)
    for ci in range(8):
        out[ci] = h[:, ci * 128:(ci + 1) * 128]


def _a2_body(agg2, cntp, h1cm, wl, wr, b, z_ref, st_ref):
    i = pl.program_id(0)
    cnt = cntp[0, :, 0:1] + cntp[1, :, 0:1]
    inv = 1.0 / jnp.maximum(cnt, 1.0)
    sagg = jnp.concatenate([agg2[ci] for ci in range(8)], axis=1)
    hc = jnp.concatenate([h1cm[ci] for ci in range(8)], axis=1)
    z = (jnp.dot(sagg * inv, wl[...], preferred_element_type=_f32)
         + jnp.dot(hc, wr[...], preferred_element_type=_f32)
         + b[...])
    z_ref[...] = z

    @pl.when(i == 0)
    def _():
        st_ref[...] = jnp.zeros_like(st_ref)

    rid = lax.broadcasted_iota(jnp.int32, (RB, 1), 0) + i * RB
    zm = jnp.where(rid < N, z, 0.0)
    st_ref[0:1, :] += jnp.sum(zm, axis=0, keepdims=True)
    st_ref[1:2, :] += jnp.sum(zm * zm, axis=0, keepdims=True)


def _bc_body(z, st, g, be, wf, bf, wf1, bf1, out):
    scale, shift = _bn_coefs(st, g, be)
    h2 = jnp.maximum(z[...] * scale + shift, 0.0)
    h3 = jnp.maximum(
        jnp.dot(h2, wf[...], preferred_element_type=_f32) + bf[...], 0.0)
    out[...] = jnp.dot(h3, wf1[...], preferred_element_type=_f32) + bf1[...]


def _full(shape):
    return pl.BlockSpec(shape, lambda i: tuple(0 for _ in shape))


_a1 = pl.pallas_call(
    _a1_body,
    grid=(GRID,),
    in_specs=[
        pl.BlockSpec((2, RB, D_IN), lambda i: (0, i, 0)),
        pl.BlockSpec((2, RB, 128), lambda i: (0, i, 0)),
        pl.BlockSpec((RB, D_IN), lambda i: (i, 0)),
        _full((D_IN, D_H)),
        _full((D_IN, D_H)),
        _full((1, D_H)),
    ],
    out_specs=[
        pl.BlockSpec((RB, D_H), lambda i: (i, 0)),
        pl.BlockSpec((8, D_H), lambda i: (0, 0)),
    ],
    out_shape=[
        jax.ShapeDtypeStruct((N, D_H), _f32),
        jax.ShapeDtypeStruct((8, D_H), _f32),
    ],
)

_b1 = pl.pallas_call(
    _b1_body,
    grid=(GRID,),
    in_specs=[
        pl.BlockSpec((RB, D_H), lambda i: (i, 0)),
        _full((8, D_H)),
        _full((1, D_H)),
        _full((1, D_H)),
    ],
    out_specs=pl.BlockSpec((8, RB, 128), lambda i: (0, i, 0)),
    out_shape=jax.ShapeDtypeStruct((8, N, 128), _f32),
)

_a2 = pl.pallas_call(
    _a2_body,
    grid=(GRID,),
    in_specs=[
        pl.BlockSpec((8, RB, 128), lambda i: (0, i, 0)),
        pl.BlockSpec((2, RB, 128), lambda i: (0, i, 0)),
        pl.BlockSpec((8, RB, 128), lambda i: (0, i, 0)),
        _full((D_H, D_H)),
        _full((D_H, D_H)),
        _full((1, D_H)),
    ],
    out_specs=[
        pl.BlockSpec((RB, D_H), lambda i: (i, 0)),
        pl.BlockSpec((8, D_H), lambda i: (0, 0)),
    ],
    out_shape=[
        jax.ShapeDtypeStruct((N, D_H), _f32),
        jax.ShapeDtypeStruct((8, D_H), _f32),
    ],
)

_bc = pl.pallas_call(
    _bc_body,
    grid=(GRID,),
    in_specs=[
        pl.BlockSpec((RB, D_H), lambda i: (i, 0)),
        _full((8, D_H)),
        _full((1, D_H)),
        _full((1, D_H)),
        _full((D_H, D_FC)),
        _full((1, D_FC)),
        _full((D_FC, D_OUT)),
        _full((1, D_OUT)),
    ],
    out_specs=pl.BlockSpec((RB, D_OUT), lambda i: (i, 0)),
    out_shape=jax.ShapeDtypeStruct((N, D_OUT), _f32),
)


def kernel(x, edge_index, batch, W1l, b1l, W1r, g1, be1,
           W2l, b2l, W2r, g2, be2, Wf, bf, Wf1, bf1):
    src = edge_index[0].astype(jnp.int32)
    dst = edge_index[1].astype(jnp.int32)

    pad1 = NB1 * K - E // 32
    srcI1 = jnp.concatenate(
        [src.reshape(32, -1), jnp.zeros((32, pad1), jnp.int32)],
        axis=1).reshape(32 * NB1, K)
    dstI1 = jnp.concatenate(
        [dst.reshape(32, -1), jnp.full((32, pad1), N, jnp.int32)],
        axis=1).reshape(32 * NB1, K)

    pad2 = NB2 * K - E // 16
    src16 = jnp.concatenate(
        [src.reshape(16, -1), jnp.zeros((16, pad2), jnp.int32)], axis=1)
    ci_off = (jnp.arange(8) * N).astype(jnp.int32)
    srcI2 = (src16[None] + ci_off[:, None, None]).reshape(8 * 16 * NB2, K)
    dstI2 = jnp.concatenate(
        [dst.reshape(16, -1), jnp.full((16, pad2), N, jnp.int32)],
        axis=1).reshape(16 * NB2, K)

    ones = jnp.ones((K, 128), _f32)
    zrow = jnp.zeros((RPT, 128), _f32)

    aggpf, cntpf = _sc_l1(x, srcI1, dstI1, ones, zrow)
    aggp = aggpf.reshape(2, NP, D_IN)
    cntp = cntpf.reshape(2, NP, 128)
    z1, st1 = _a1(aggp, cntp, x, W1l, W1r, b1l.reshape(1, -1))
    h1cm = _b1(z1, st1, g1.reshape(1, -1), be1.reshape(1, -1))
    h1f = h1cm.reshape(8 * N, 128)
    agg2 = _sc_l2(h1f, srcI2, dstI2, zrow).reshape(8, NP, 128)
    z2, st2 = _a2(agg2, cntp, h1cm, W2l, W2r, b2l.reshape(1, -1))
    out = _bc(z2, st2, g2.reshape(1, -1), be2.reshape(1, -1),
              Wf, bf.reshape(1, -1), Wf1, bf1.reshape(1, -1))
    return out


# speedup vs baseline: 2.9607x; 1.0381x over previous
import jax
import jax.numpy as jnp
from jax import lax
from jax.experimental import pallas as pl
from jax.experimental.pallas import tpu as pltpu
from jax.experimental.pallas import tpu_sc as plsc

N = 10000
E = 320000
D_IN = 128
D_H = 1024
D_FC = 512
D_OUT = 128
EPS_BN = 1e-5

NP = 10112
RPT = NP // 16
K = 128
SUP = 16
NB1 = 80
NB2 = 160
RB = 512
GRID = (N + RB - 1) // RB

_f32 = jnp.float32
_mesh = plsc.VectorSubcoreMesh(core_axis_name="c", subcore_axis_name="s")


def _sc_l1_body(x_hbm, srcI, dstI, ones_hbm, zrow_hbm,
                aggp_hbm, cntp_hbm,
                srcb, dstb, rowbuf0, rowbuf1, aggS,
                semg0, semg1, sems0, sems1):
    c = lax.axis_index("c")
    s = lax.axis_index("s")
    base = s * RPT
    irow0 = (c * 16 + s) * NB1
    bufs = (rowbuf0, rowbuf1)
    semg = (semg0, semg1)
    sems = (sems0, sems1)
    pltpu.sync_copy(zrow_hbm, aggS.at[pl.ds(base, RPT)])
    plsc.subcore_barrier()

    def sup_body(g, carry):
        pltpu.sync_copy(srcI.at[pl.ds(irow0 + g * SUP, SUP)], srcb)
        pltpu.sync_copy(dstI.at[pl.ds(irow0 + g * SUP, SUP)], dstb)
        gd = [None, None]
        sd = [None, None]
        gd[0] = pltpu.async_copy(x_hbm.at[srcb.at[0]], bufs[0], semg[0])
        for bi in range(SUP):
            cur = bi % 2
            nxt = (bi + 1) % 2
            if bi + 1 < SUP:
                if sd[nxt] is not None:
                    sd[nxt].wait()
                gd[nxt] = pltpu.async_copy(
                    x_hbm.at[srcb.at[bi + 1]], bufs[nxt], semg[nxt])
            gd[cur].wait()
            sd[cur] = pltpu.async_copy(
                bufs[cur], aggS.at[dstb.at[bi]], sems[cur], add=True)
        sd[0].wait()
        sd[1].wait()
        return carry

    lax.fori_loop(0, NB1 // SUP, sup_body, 0)
    plsc.subcore_barrier()
    pltpu.sync_copy(aggS.at[pl.ds(base, RPT)],
                    aggp_hbm.at[pl.ds(c * NP + base, RPT)])
    pltpu.sync_copy(zrow_hbm, aggS.at[pl.ds(base, RPT)])
    pltpu.sync_copy(ones_hbm, rowbuf1)
    plsc.subcore_barrier()

    def cnt_body(g, carry):
        pltpu.sync_copy(dstI.at[pl.ds(irow0 + g * SUP, SUP)], dstb)
        ds_ = [pltpu.async_copy(rowbuf1, aggS.at[dstb.at[bi]],
                                sems[bi % 2], add=True)
               for bi in range(SUP)]
        for d in ds_:
            d.wait()
        return carry

    lax.fori_loop(0, NB1 // SUP, cnt_body, 0)
    plsc.subcore_barrier()
    pltpu.sync_copy(aggS.at[pl.ds(base, RPT)],
                    cntp_hbm.at[pl.ds(c * NP + base, RPT)])


_sc_l1 = pl.kernel(
    _sc_l1_body,
    out_type=(jax.ShapeDtypeStruct((2 * NP, D_IN), _f32),
              jax.ShapeDtypeStruct((2 * NP, 128), _f32)),
    mesh=_mesh,
    scratch_types=[
        pltpu.VMEM((SUP, K), jnp.int32),
        pltpu.VMEM((SUP, K), jnp.int32),
        pltpu.VMEM((K, D_IN), _f32),
        pltpu.VMEM((K, 128), _f32),
        pltpu.VMEM_SHARED((NP, D_IN), _f32),
        pltpu.SemaphoreType.DMA,
        pltpu.SemaphoreType.DMA,
        pltpu.SemaphoreType.DMA,
        pltpu.SemaphoreType.DMA,
    ],
)


def _sc_l2_body(h1f_hbm, srcI2, dstI2, zrow_hbm,
                agg2_hbm,
                srcb, dstb, rowbuf0, rowbuf1, aggS,
                semg0, semg1, sems0, sems1):
    c = lax.axis_index("c")
    s = lax.axis_index("s")
    base = s * RPT
    bufs = (rowbuf0, rowbuf1)
    semg = (semg0, semg1)
    sems = (sems0, sems1)
    for j in range(4):
        ci = 2 * j + c
        irow0 = (ci * 16 + s) * NB2
        drow0 = s * NB2
        pltpu.sync_copy(zrow_hbm, aggS.at[pl.ds(base, RPT)])
        plsc.subcore_barrier()

        def sup_body(g, carry):
            pltpu.sync_copy(srcI2.at[pl.ds(irow0 + g * SUP, SUP)], srcb)
            pltpu.sync_copy(dstI2.at[pl.ds(drow0 + g * SUP, SUP)], dstb)
            gd = [None, None]
            sd = [None, None]
            gd[0] = pltpu.async_copy(h1f_hbm.at[srcb.at[0]], bufs[0], semg[0])
            for bi in range(SUP):
                cur = bi % 2
                nxt = (bi + 1) % 2
                if bi + 1 < SUP:
                    if sd[nxt] is not None:
                        sd[nxt].wait()
                    gd[nxt] = pltpu.async_copy(
                        h1f_hbm.at[srcb.at[bi + 1]], bufs[nxt], semg[nxt])
                gd[cur].wait()
                sd[cur] = pltpu.async_copy(
                    bufs[cur], aggS.at[dstb.at[bi]], sems[cur], add=True)
            sd[0].wait()
            sd[1].wait()
            return carry

        lax.fori_loop(0, NB2 // SUP, sup_body, 0)
        plsc.subcore_barrier()
        pltpu.sync_copy(aggS.at[pl.ds(base, RPT)],
                        agg2_hbm.at[pl.ds(ci * NP + base, RPT)])


_sc_l2 = pl.kernel(
    _sc_l2_body,
    out_type=jax.ShapeDtypeStruct((8 * NP, 128), _f32),
    mesh=_mesh,
    scratch_types=[
        pltpu.VMEM((SUP, K), jnp.int32),
        pltpu.VMEM((SUP, K), jnp.int32),
        pltpu.VMEM((K, 128), _f32),
        pltpu.VMEM((K, 128), _f32),
        pltpu.VMEM_SHARED((NP, 128), _f32),
        pltpu.SemaphoreType.DMA,
        pltpu.SemaphoreType.DMA,
        pltpu.SemaphoreType.DMA,
        pltpu.SemaphoreType.DMA,
    ],
)


def _a1_body(aggp, cntp, x, wl, wr, b, z_ref, st_ref):
    i = pl.program_id(0)
    agg = aggp[0] + aggp[1]
    cnt = cntp[0, :, 0:1] + cntp[1, :, 0:1]
    inv = 1.0 / jnp.maximum(cnt, 1.0)
    z = (jnp.dot(agg * inv, wl[...], preferred_element_type=_f32)
         + jnp.dot(x[...], wr[...], preferred_element_type=_f32)
         + b[...])
    z_ref[...] = z

    @pl.when(i == 0)
    def _():
        st_ref[...] = jnp.zeros_like(st_ref)

    rid = lax.broadcasted_iota(jnp.int32, (RB, 1), 0) + i * RB
    zm = jnp.where(rid < N, z, 0.0)
    st_ref[0:1, :] += jnp.sum(zm, axis=0, keepdims=True)
    st_ref[1:2, :] += jnp.sum(zm * zm, axis=0, keepdims=True)


def _bn_coefs(st, g, be):
    mu = st[0:1, :] * (1.0 / N)
    var = st[1:2, :] * (1.0 / N) - mu * mu
    scale = g[...] * lax.rsqrt(var + EPS_BN)
    shift = be[...] - mu * scale
    return scale, shift


def _b1_body(z, st, g, be, out):
    scale, shift = _bn_coefs(st, g, be)
    h = jnp.maximum(z[...] * scale + shift, 0.0)
    for ci in range(8):
        out[ci] = h[:, ci * 128:(ci + 1) * 128]


def _a2_body(agg2, cntp, h1cm, wl, wr, b, z_ref, st_ref):
    i = pl.program_id(0)
    cnt = cntp[0, :, 0:1] + cntp[1, :, 0:1]
    inv = 1.0 / jnp.maximum(cnt, 1.0)
    sagg = jnp.concatenate([agg2[ci] for ci in range(8)], axis=1)
    hc = jnp.concatenate([h1cm[ci] for ci in range(8)], axis=1)
    z = (jnp.dot(sagg * inv, wl[...], preferred_element_type=_f32)
         + jnp.dot(hc, wr[...], preferred_element_type=_f32)
         + b[...])
    z_ref[...] = z

    @pl.when(i == 0)
    def _():
        st_ref[...] = jnp.zeros_like(st_ref)

    rid = lax.broadcasted_iota(jnp.int32, (RB, 1), 0) + i * RB
    zm = jnp.where(rid < N, z, 0.0)
    st_ref[0:1, :] += jnp.sum(zm, axis=0, keepdims=True)
    st_ref[1:2, :] += jnp.sum(zm * zm, axis=0, keepdims=True)


def _bc_body(z, st, g, be, wf, bf, wf1, bf1, out):
    scale, shift = _bn_coefs(st, g, be)
    h2 = jnp.maximum(z[...] * scale + shift, 0.0)
    h3 = jnp.maximum(
        jnp.dot(h2, wf[...], preferred_element_type=_f32) + bf[...], 0.0)
    out[...] = jnp.dot(h3, wf1[...], preferred_element_type=_f32) + bf1[...]


def _full(shape):
    return pl.BlockSpec(shape, lambda i: tuple(0 for _ in shape))


_a1 = pl.pallas_call(
    _a1_body,
    grid=(GRID,),
    in_specs=[
        pl.BlockSpec((2, RB, D_IN), lambda i: (0, i, 0)),
        pl.BlockSpec((2, RB, 128), lambda i: (0, i, 0)),
        pl.BlockSpec((RB, D_IN), lambda i: (i, 0)),
        _full((D_IN, D_H)),
        _full((D_IN, D_H)),
        _full((1, D_H)),
    ],
    out_specs=[
        pl.BlockSpec((RB, D_H), lambda i: (i, 0)),
        pl.BlockSpec((8, D_H), lambda i: (0, 0)),
    ],
    out_shape=[
        jax.ShapeDtypeStruct((N, D_H), _f32),
        jax.ShapeDtypeStruct((8, D_H), _f32),
    ],
)

_b1 = pl.pallas_call(
    _b1_body,
    grid=(GRID,),
    in_specs=[
        pl.BlockSpec((RB, D_H), lambda i: (i, 0)),
        _full((8, D_H)),
        _full((1, D_H)),
        _full((1, D_H)),
    ],
    out_specs=pl.BlockSpec((8, RB, 128), lambda i: (0, i, 0)),
    out_shape=jax.ShapeDtypeStruct((8, N, 128), _f32),
)

_a2 = pl.pallas_call(
    _a2_body,
    grid=(GRID,),
    in_specs=[
        pl.BlockSpec((8, RB, 128), lambda i: (0, i, 0)),
        pl.BlockSpec((2, RB, 128), lambda i: (0, i, 0)),
        pl.BlockSpec((8, RB, 128), lambda i: (0, i, 0)),
        _full((D_H, D_H)),
        _full((D_H, D_H)),
        _full((1, D_H)),
    ],
    out_specs=[
        pl.BlockSpec((RB, D_H), lambda i: (i, 0)),
        pl.BlockSpec((8, D_H), lambda i: (0, 0)),
    ],
    out_shape=[
        jax.ShapeDtypeStruct((N, D_H), _f32),
        jax.ShapeDtypeStruct((8, D_H), _f32),
    ],
)

_bc = pl.pallas_call(
    _bc_body,
    grid=(GRID,),
    in_specs=[
        pl.BlockSpec((RB, D_H), lambda i: (i, 0)),
        _full((8, D_H)),
        _full((1, D_H)),
        _full((1, D_H)),
        _full((D_H, D_FC)),
        _full((1, D_FC)),
        _full((D_FC, D_OUT)),
        _full((1, D_OUT)),
    ],
    out_specs=pl.BlockSpec((RB, D_OUT), lambda i: (i, 0)),
    out_shape=jax.ShapeDtypeStruct((N, D_OUT), _f32),
)


def kernel(x, edge_index, batch, W1l, b1l, W1r, g1, be1,
           W2l, b2l, W2r, g2, be2, Wf, bf, Wf1, bf1):
    src = edge_index[0].astype(jnp.int32)
    dst = edge_index[1].astype(jnp.int32)

    pad1 = NB1 * K - E // 32
    srcI1 = jnp.concatenate(
        [src.reshape(32, -1), jnp.zeros((32, pad1), jnp.int32)],
        axis=1).reshape(32 * NB1, K)
    dstI1 = jnp.concatenate(
        [dst.reshape(32, -1), jnp.full((32, pad1), N, jnp.int32)],
        axis=1).reshape(32 * NB1, K)

    pad2 = NB2 * K - E // 16
    src16 = jnp.concatenate(
        [src.reshape(16, -1), jnp.zeros((16, pad2), jnp.int32)], axis=1)
    ci_off = (jnp.arange(8) * N).astype(jnp.int32)
    srcI2 = (src16[None] + ci_off[:, None, None]).reshape(8 * 16 * NB2, K)
    dstI2 = jnp.concatenate(
        [dst.reshape(16, -1), jnp.full((16, pad2), N, jnp.int32)],
        axis=1).reshape(16 * NB2, K)

    ones = jnp.ones((K, 128), _f32)
    zrow = jnp.zeros((RPT, 128), _f32)

    aggpf, cntpf = _sc_l1(x, srcI1, dstI1, ones, zrow)
    aggp = aggpf.reshape(2, NP, D_IN)
    cntp = cntpf.reshape(2, NP, 128)
    z1, st1 = _a1(aggp, cntp, x, W1l, W1r, b1l.reshape(1, -1))
    h1cm = _b1(z1, st1, g1.reshape(1, -1), be1.reshape(1, -1))
    h1f = h1cm.reshape(8 * N, 128)
    agg2 = _sc_l2(h1f, srcI2, dstI2, zrow).reshape(8, NP, 128)
    z2, st2 = _a2(agg2, cntp, h1cm, W2l, W2r, b2l.reshape(1, -1))
    out = _bc(z2, st2, g2.reshape(1, -1), be2.reshape(1, -1),
              Wf, bf.reshape(1, -1), Wf1, bf1.reshape(1, -1))
    return out
